# baseline (speedup 1.0000x reference)
import math

import jax
import jax.numpy as jnp
import numpy as np
from jax.experimental import pallas as pl
from jax.experimental.pallas import tpu as pltpu

_N = 512
_H = 128
_F = 128
_R = 50
_L = 6
_CUT = 10.0
_ZPAD = 128
_NMOL = 16

_D0 = -3.0
_LDOM = 16.0
_M = 1024
_K = 128

def _dct_consts():
    m = np.arange(_M)
    dg = _D0 + _LDOM * (m + 0.5) / _M
    cen = np.linspace(0.0, _CUT, _R)
    w = _CUT / _R
    rbf_gridT = np.exp(-((cen[:, None] - dg[None, :]) ** 2) / (2 * w * w))
    kk = np.arange(_K)
    PT = (np.cos(np.pi * (m[:, None] + 0.5) * kk[None, :] / _M) * (2.0 / _M))
    PT[:, 0] *= 0.5
    return (jnp.asarray(rbf_gridT, jnp.float32),
            jnp.asarray(PT, jnp.float32))

_RBF_GRIDT, _PT = _dct_consts()


def _schnet_body(az_ref, pos_ref, posT_ref, bat_ref, embT_ref,
                 rbfgT_ref, PT_ref,
                 fW1T_ref, fb1_ref, fW2T_ref, fb2_ref,
                 dW1T_ref, db1_ref, dW2T_ref, db2_ref,
                 oW1T_ref, ob1_ref, oW2T_ref, ob2_ref,
                 out_ref,
                 D_ref, CUT_ref, xT_ref, yT_ref, accT_ref):
    f32 = jnp.float32
    pos = pos_ref[:, :]
    posT = posT_ref[:, :]
    sq = jnp.sum(pos * pos, axis=1, keepdims=True)
    sqT = jnp.sum(posT * posT, axis=0, keepdims=True)
    g = jnp.dot(pos, posT, preferred_element_type=f32)
    d = jnp.sqrt(jnp.maximum(sq + sqT - 2.0 * g, 0.0))
    D_ref[:, :] = d
    ii = jax.lax.broadcasted_iota(jnp.int32, (_N, _N), 0)
    jj = jax.lax.broadcasted_iota(jnp.int32, (_N, _N), 1)
    valid = (ii != jj) & (d < _CUT)
    CUT_ref[:, :] = (0.5 * (jnp.cos(d * (math.pi / _CUT)) + 1.0)
                     * valid.astype(f32))

    zio = jax.lax.broadcasted_iota(jnp.int32, (_ZPAD, _N), 0)
    oh = (zio == az_ref[:, :]).astype(f32)
    xT_ref[:, :] = jnp.dot(embT_ref[:, :], oh, preferred_element_type=f32)

    kcol = (jax.lax.broadcasted_iota(jnp.int32, (_K, 1), 0).astype(f32)
            * (math.pi / _LDOM))

    for l in range(_L):
        g1 = (jnp.dot(fW1T_ref[l], rbfgT_ref[:, :],
                      preferred_element_type=f32) + fb1_ref[l])
        g1 = g1 * jax.nn.sigmoid(g1)
        GT = (jnp.dot(fW2T_ref[l], g1, preferred_element_type=f32)
              + fb2_ref[l])
        CT = jnp.dot(GT, PT_ref[:, :], preferred_element_type=f32)

        yT_ref[:, :] = (jnp.dot(dW1T_ref[l], xT_ref[:, :],
                                preferred_element_type=f32) + db1_ref[l])
        accT_ref[:, :] = jnp.zeros((_F, _N), f32)

        def body(ib, carry):
            dblk = D_ref[pl.ds(ib * 128, 128), :]
            cblk = CUT_ref[pl.ds(ib * 128, 128), :]
            yblk = yT_ref[:, pl.ds(ib * 128, 128)]
            acc = jnp.zeros((_F, _N), f32)
            for r in range(128):
                d_row = dblk[r:r + 1, :]
                cut_row = cblk[r:r + 1, :]
                cosT = jnp.cos(kcol * (d_row - _D0))
                filt = jnp.dot(CT, cosT,
                               preferred_element_type=f32) * cut_row
                acc = acc + yblk[:, r:r + 1] * filt
            accT_ref[:, :] += acc
            return carry

        jax.lax.fori_loop(0, _N // 128, body, 0)
        xT_ref[:, :] = (xT_ref[:, :]
                        + jnp.dot(dW2T_ref[l], accT_ref[:, :],
                                  preferred_element_type=f32) + db2_ref[l])

    s1 = jnp.dot(oW1T_ref[:, :], xT_ref[:, :],
                 preferred_element_type=f32) + ob1_ref[:, :]
    s1 = s1 * jax.nn.sigmoid(s1)
    hT = jnp.dot(oW2T_ref[:, :], s1,
                 preferred_element_type=f32) + ob2_ref[:, :]
    mio = jax.lax.broadcasted_iota(jnp.int32, (_N, 128), 1)
    boh = (mio == bat_ref[:, :]).astype(f32)
    pooled = jnp.dot(hT, boh, preferred_element_type=f32)
    out_ref[:, :] = pooled[:, :_NMOL]


def kernel(atomic_numbers, positions, batch, emb, fW1, fb1, fW2, fb2,
           dW1, db1, dW2, db2, oW1, ob1, oW2, ob2):
    f32 = jnp.float32
    az = atomic_numbers.astype(jnp.int32).reshape(1, _N)
    pos = positions.astype(f32)
    posT = pos.T
    bat = batch.astype(jnp.int32).reshape(_N, 1)
    embT = jnp.zeros((_H, _ZPAD), f32).at[:, :emb.shape[0]].set(emb.T)
    fW1T = fW1.transpose(0, 2, 1)
    fW2T = fW2.transpose(0, 2, 1)
    dW1T = dW1.transpose(0, 2, 1)
    dW2T = dW2.transpose(0, 2, 1)
    oW1T = oW1.T
    oW2T = oW2.T
    fb1c = fb1[:, :, None]
    fb2c = fb2[:, :, None]
    db1c = db1[:, :, None]
    db2c = db2[:, :, None]
    ob1c = ob1[:, None]
    ob2c = ob2[:, None]

    pooled = pl.pallas_call(
        _schnet_body,
        out_shape=jax.ShapeDtypeStruct((1, _NMOL), f32),
        scratch_shapes=[
            pltpu.VMEM((_N, _N), f32),
            pltpu.VMEM((_N, _N), f32),
            pltpu.VMEM((_H, _N), f32),
            pltpu.VMEM((_F, _N), f32),
            pltpu.VMEM((_F, _N), f32),
        ],
    )(az, pos, posT, bat, embT, _RBF_GRIDT, _PT,
      fW1T, fb1c, fW2T, fb2c, dW1T, db1c, dW2T, db2c,
      oW1T, ob1c, oW2T, ob2c)
    return pooled.reshape(_NMOL, 1)


# speedup vs baseline: 3.5143x; 3.5143x over previous
import math

import jax
import jax.numpy as jnp
import numpy as np
from jax.experimental import pallas as pl
from jax.experimental.pallas import tpu as pltpu

_N = 512
_H = 128
_F = 128
_R = 50
_L = 6
_CUT = 10.0
_ZPAD = 128
_NMOL = 16


def _schnet_body(az_ref, pos_ref, posT_ref, bat_ref, embT_ref, cen_ref,
                 fW1T_ref, fb1_ref, fW2T_ref, fb2_ref,
                 dW1T_ref, db1_ref, dW2T_ref, db2_ref,
                 oW1T_ref, ob1_ref, oW2T_ref, ob2_ref,
                 out_ref,
                 D_ref, CUT_ref, xT_ref, yT_ref, accT_ref):
    f32 = jnp.float32
    pos = pos_ref[:, :]
    posT = posT_ref[:, :]
    sq = jnp.sum(pos * pos, axis=1, keepdims=True)
    sqT = jnp.sum(posT * posT, axis=0, keepdims=True)
    g = jnp.dot(pos, posT, preferred_element_type=f32)
    d = jnp.sqrt(jnp.maximum(sq + sqT - 2.0 * g, 0.0))
    D_ref[:, :] = d
    ii = jax.lax.broadcasted_iota(jnp.int32, (_N, _N), 0)
    jj = jax.lax.broadcasted_iota(jnp.int32, (_N, _N), 1)
    valid = (ii != jj) & (d < _CUT)
    CUT_ref[:, :] = (0.5 * (jnp.cos(d * (math.pi / _CUT)) + 1.0)
                     * valid.astype(f32))

    zio = jax.lax.broadcasted_iota(jnp.int32, (_ZPAD, _N), 0)
    oh = (zio == az_ref[:, :]).astype(f32)
    xT_ref[:, :] = jnp.dot(embT_ref[:, :], oh, preferred_element_type=f32)

    centers = cen_ref[:, :]
    inv2w2 = 1.0 / (2.0 * (_CUT / _R) ** 2)
    bf16 = jnp.bfloat16

    for l in range(_L):
        fW1Tb = fW1T_ref[l].astype(bf16)
        fW2Tb = fW2T_ref[l].astype(bf16)
        fb1 = fb1_ref[l]
        fb2 = fb2_ref[l]
        yT_ref[:, :] = (jnp.dot(dW1T_ref[l], xT_ref[:, :],
                                preferred_element_type=f32) + db1_ref[l])
        accT_ref[:, :] = jnp.zeros((_F, _N), f32)

        def body(ib, carry):
            dblk = D_ref[pl.ds(ib * 128, 128), :]
            cblk = CUT_ref[pl.ds(ib * 128, 128), :]
            yblk = yT_ref[:, pl.ds(ib * 128, 128)]
            acc = jnp.zeros((_F, _N), f32)
            for r in range(128):
                d_row = dblk[r:r + 1, :]
                cut_row = cblk[r:r + 1, :]
                rbfT = jnp.exp(-((d_row - centers) ** 2)
                               * inv2w2).astype(bf16)
                h1 = jnp.dot(fW1Tb, rbfT, preferred_element_type=f32) + fb1
                h1 = (h1 * jax.nn.sigmoid(h1)).astype(bf16)
                filt = (jnp.dot(fW2Tb, h1, preferred_element_type=f32)
                        + fb2) * cut_row
                acc = acc + yblk[:, r:r + 1] * filt
            accT_ref[:, :] += acc
            return carry

        jax.lax.fori_loop(0, _N // 128, body, 0)
        xT_ref[:, :] = (xT_ref[:, :]
                        + jnp.dot(dW2T_ref[l], accT_ref[:, :],
                                  preferred_element_type=f32) + db2_ref[l])

    s1 = jnp.dot(oW1T_ref[:, :], xT_ref[:, :],
                 preferred_element_type=f32) + ob1_ref[:, :]
    s1 = s1 * jax.nn.sigmoid(s1)
    hT = jnp.dot(oW2T_ref[:, :], s1,
                 preferred_element_type=f32) + ob2_ref[:, :]
    mio = jax.lax.broadcasted_iota(jnp.int32, (_N, 128), 1)
    boh = (mio == bat_ref[:, :]).astype(f32)
    pooled = jnp.dot(hT, boh, preferred_element_type=f32)
    out_ref[:, :] = pooled[:, :_NMOL]


def kernel(atomic_numbers, positions, batch, emb, fW1, fb1, fW2, fb2,
           dW1, db1, dW2, db2, oW1, ob1, oW2, ob2):
    f32 = jnp.float32
    az = atomic_numbers.astype(jnp.int32).reshape(1, _N)
    pos = positions.astype(f32)
    posT = pos.T
    bat = batch.astype(jnp.int32).reshape(_N, 1)
    embT = jnp.zeros((_H, _ZPAD), f32).at[:, :emb.shape[0]].set(emb.T)
    cen = jnp.linspace(0.0, _CUT, _R).astype(f32).reshape(_R, 1)
    fW1T = fW1.transpose(0, 2, 1)
    fW2T = fW2.transpose(0, 2, 1)
    dW1T = dW1.transpose(0, 2, 1)
    dW2T = dW2.transpose(0, 2, 1)
    oW1T = oW1.T
    oW2T = oW2.T
    fb1c = fb1[:, :, None]
    fb2c = fb2[:, :, None]
    db1c = db1[:, :, None]
    db2c = db2[:, :, None]
    ob1c = ob1[:, None]
    ob2c = ob2[:, None]

    pooled = pl.pallas_call(
        _schnet_body,
        out_shape=jax.ShapeDtypeStruct((1, _NMOL), f32),
        scratch_shapes=[
            pltpu.VMEM((_N, _N), f32),
            pltpu.VMEM((_N, _N), f32),
            pltpu.VMEM((_H, _N), f32),
            pltpu.VMEM((_F, _N), f32),
            pltpu.VMEM((_F, _N), f32),
        ],
    )(az, pos, posT, bat, embT, cen,
      fW1T, fb1c, fW2T, fb2c, dW1T, db1c, dW2T, db2c,
      oW1T, ob1c, oW2T, ob2c)
    return pooled.reshape(_NMOL, 1)
